# Initial kernel scaffold; baseline (speedup 1.0000x reference)
import jax
import jax.numpy as jnp
from jax.experimental import pallas as pl


def kernel(x, params, edge_index):
    raise NotImplementedError("write your pallas kernel here")


# speedup vs baseline: 3.9203x; 3.9203x over previous
import functools

import jax
import jax.numpy as jnp
from jax import lax
from jax.experimental import pallas as pl
from jax.experimental.pallas import tpu as pltpu
from jax.experimental.pallas import tpu_sc as plsc

_NC = 2
_NS = 16
_NW = _NC * _NS
_D = 128
_EPS = 1e-5
_CHUNK = 128
_N_ACC = 10112
_BE = 4096


def _gelu(t):
    return 0.5 * t * (1.0 + lax.erf(t * 0.7071067811865476))


def _ln_rows(t, g, b):
    m = jnp.mean(t, axis=-1, keepdims=True)
    v = jnp.mean((t - m) ** 2, axis=-1, keepdims=True)
    return g * (t - m) * lax.rsqrt(v + _EPS) + b


def _mm(a, w):
    return lax.dot_general(a, w, (((1,), (1,)), ((), ())),
                           preferred_element_type=jnp.float32)


def _node_mlp_kernel(x_ref, w1_ref, g1_ref, be1_ref, w2_ref, g2_ref, be2_ref,
                     o_ref):
    h = _mm(x_ref[...], w1_ref[...])
    m = jnp.mean(h, axis=0)
    v = jnp.mean((h - m) ** 2, axis=0)
    h = _gelu(g1_ref[...] * (h - m) * lax.rsqrt(v + _EPS) + be1_ref[...])
    h2 = _mm(h, w2_ref[...])
    m2 = jnp.mean(h2, axis=0)
    v2 = jnp.mean((h2 - m2) ** 2, axis=0)
    o_ref[...] = _gelu(g2_ref[...] * (h2 - m2) * lax.rsqrt(v2 + _EPS)
                       + be2_ref[...])


def _node_mlp(x, p):
    n = x.shape[0]
    return pl.pallas_call(
        _node_mlp_kernel,
        out_shape=jax.ShapeDtypeStruct((n, _D), jnp.float32),
    )(x, p['emb_W1'], p['emb_g1'].reshape(1, -1), p['emb_be1'].reshape(1, -1),
      p['emb_W2'], p['emb_g2'].reshape(1, -1), p['emb_be2'].reshape(1, -1))


def _proj_kernel(h_ref, wi_ref, wj_ref, oi_ref, oj_ref):
    oi_ref[...] = _mm(h_ref[...], wi_ref[...])
    oj_ref[...] = _mm(h_ref[...], wj_ref[...])


def _proj(h, wi, wj):
    n = h.shape[0]
    sh = jax.ShapeDtypeStruct((n, _D), jnp.float32)
    return pl.pallas_call(_proj_kernel, out_shape=(sh, sh))(h, wi, wj)


def _mm2_kernel(a_ref, b_ref, w_ref, o_ref):
    o_ref[...] = _mm(a_ref[...] + b_ref[...], w_ref[...])


def _mm2(a, b, w):
    n = a.shape[0]
    return pl.pallas_call(
        _mm2_kernel, out_shape=jax.ShapeDtypeStruct((n, _D), jnp.float32),
    )(a, b, w)


def _update_kernel(n, h_ref, ag_ref, wu1_ref, wu2_ref, bu_ref, gu_ref,
                   beu_ref, o_ref):
    h = h_ref[...]
    ag = ag_ref[...]
    agg = ag[0, :n] + ag[1, :n]
    t = _mm(h, wu1_ref[...]) + _mm(agg, wu2_ref[...]) + bu_ref[...]
    o_ref[...] = h + _gelu(_ln_rows(t, gu_ref[...], beu_ref[...]))


def _update(h, agg2, lp):
    n = h.shape[0]
    return pl.pallas_call(
        functools.partial(_update_kernel, n),
        out_shape=jax.ShapeDtypeStruct((n, _D), jnp.float32),
    )(h, agg2, lp['Wu'][:, :_D], lp['Wu'][:, _D:],
      lp['bu'].reshape(1, -1), lp['gu'].reshape(1, -1),
      lp['beu'].reshape(1, -1))


def _msg_kernel(a_ref, b_ref, bm_ref, gm_ref, bem_ref, o_ref):
    t = a_ref[...] + b_ref[...] + bm_ref[...]
    o_ref[...] = _gelu(_ln_rows(t, gm_ref[...], bem_ref[...]))


def _msg(ga, gb, lp):
    e = ga.shape[0]
    grid = e // _BE
    blk = pl.BlockSpec((_BE, _D), lambda i: (i, 0))
    par = pl.BlockSpec((1, _D), lambda i: (0, 0))
    return pl.pallas_call(
        _msg_kernel,
        grid=(grid,),
        in_specs=[blk, blk, par, par, par],
        out_specs=blk,
        out_shape=jax.ShapeDtypeStruct((e, _D), jnp.float32),
    )(ga, gb, lp['bm'].reshape(1, -1), lp['gm'].reshape(1, -1),
      lp['bem'].reshape(1, -1))


def _stats_kernel(a_ref, b_ref, o_ref):
    q = a_ref[...] - b_ref[...]
    blk = jnp.concatenate(
        [jnp.sum(q, axis=0, keepdims=True),
         jnp.sum(q * q, axis=0, keepdims=True)], axis=0)

    @pl.when(pl.program_id(0) == 0)
    def _init():
        o_ref[...] = jnp.zeros_like(o_ref)

    o_ref[...] += blk


def _stats(ga, gb):
    e = ga.shape[0]
    blk = pl.BlockSpec((_BE, _D), lambda i: (i, 0))
    return pl.pallas_call(
        _stats_kernel,
        grid=(e // _BE,),
        in_specs=[blk, blk],
        out_specs=pl.BlockSpec((2, _D), lambda i: (0, 0)),
        out_shape=jax.ShapeDtypeStruct((2, _D), jnp.float32),
    )(ga, gb)


def _final_kernel(n_real, a_ref, b_ref, st_ref, g_ref, be_ref, w2_ref, b2_ref,
                  o_ref):
    q = a_ref[...] - b_ref[...]
    st = st_ref[...]
    m = st[0:1] * (1.0 / n_real)
    v = st[1:2] * (1.0 / n_real) - m * m
    t = _gelu(g_ref[...] * (q - m) * lax.rsqrt(v + _EPS) + be_ref[...])
    o_ref[...] = _mm(t, w2_ref[...]) + b2_ref[...]


def _final(ga, gb, st, p, n_real):
    e = ga.shape[0]
    blk = pl.BlockSpec((_BE, _D), lambda i: (i, 0))
    par = pl.BlockSpec((1, _D), lambda i: (0, 0))
    w2p = jnp.zeros((8, _D), jnp.float32).at[:3].set(p['fin_W2'])
    b2p = jnp.zeros((1, 8), jnp.float32).at[0, :3].set(p['fin_b2'])
    return pl.pallas_call(
        functools.partial(_final_kernel, float(n_real)),
        grid=(e // _BE,),
        in_specs=[blk, blk,
                  pl.BlockSpec((2, _D), lambda i: (0, 0)), par, par,
                  pl.BlockSpec((8, _D), lambda i: (0, 0)),
                  pl.BlockSpec((1, 8), lambda i: (0, 0))],
        out_specs=pl.BlockSpec((_BE, 8), lambda i: (i, 0)),
        out_shape=jax.ShapeDtypeStruct((e, 8), jnp.float32),
    )(ga, gb, st, p['fin_g1'].reshape(1, -1), p['fin_be1'].reshape(1, -1),
      w2p, b2p)


def _sc_mesh():
    return plsc.VectorSubcoreMesh(core_axis_name="c", subcore_axis_name="s",
                                  num_cores=_NC, num_subcores=_NS)


def _sc_gather2(ta, tb, ia, ib):
    e = ia.shape[0]
    per_w = e // _NW
    n_ch = per_w // _CHUNK
    sh = jax.ShapeDtypeStruct((e, _D), jnp.float32)

    @functools.partial(
        pl.kernel,
        out_type=(sh, sh),
        mesh=_sc_mesh(),
        scratch_types=[
            pltpu.VMEM((_CHUNK,), jnp.int32),
            pltpu.VMEM((_CHUNK,), jnp.int32),
            pltpu.VMEM((_CHUNK, _D), jnp.float32),
            pltpu.VMEM((_CHUNK, _D), jnp.float32),
            pltpu.SemaphoreType.DMA,
            pltpu.SemaphoreType.DMA,
        ],
    )
    def k(ta_h, tb_h, ia_h, ib_h, oa_h, ob_h, ia_v, ib_v, ra_v, rb_v, s1, s2):
        wid = lax.axis_index("s") * _NC + lax.axis_index("c")
        base_w = wid * per_w

        def body(i, carry):
            base = base_w + i * _CHUNK
            pltpu.sync_copy(ia_h.at[pl.ds(base, _CHUNK)], ia_v)
            pltpu.sync_copy(ib_h.at[pl.ds(base, _CHUNK)], ib_v)
            ca = pltpu.async_copy(ta_h.at[ia_v], ra_v, s1)
            cb = pltpu.async_copy(tb_h.at[ib_v], rb_v, s2)
            ca.wait()
            cb.wait()
            pltpu.sync_copy(ra_v, oa_h.at[pl.ds(base, _CHUNK)])
            pltpu.sync_copy(rb_v, ob_h.at[pl.ds(base, _CHUNK)])
            return carry

        lax.fori_loop(0, n_ch, body, 0)

    return k(ta, tb, ia, ib)


def _sc_scatter_add(msg, dsts, zrows):
    e = msg.shape[0]
    per_w = e // _NW
    n_ch = per_w // _CHUNK
    zc = _N_ACC // _NS

    @functools.partial(
        pl.kernel,
        out_type=jax.ShapeDtypeStruct((_NC, _N_ACC, _D), jnp.float32),
        mesh=_sc_mesh(),
        scratch_types=[
            pltpu.VMEM((_CHUNK,), jnp.int32),
            pltpu.VMEM((_CHUNK, _D), jnp.float32),
            pltpu.VMEM_SHARED((_N_ACC, _D), jnp.float32),
        ],
    )
    def k(msg_h, dst_h, z_h, out_h, idx_v, rows_v, shared):
        c = lax.axis_index("c")
        s = lax.axis_index("s")
        wid = s * _NC + c
        pltpu.sync_copy(z_h.at[pl.ds(s * zc, zc)], shared.at[pl.ds(s * zc, zc)])
        plsc.subcore_barrier()
        base_w = wid * per_w

        def body(i, carry):
            base = base_w + i * _CHUNK
            pltpu.sync_copy(dst_h.at[pl.ds(base, _CHUNK)], idx_v)
            pltpu.sync_copy(msg_h.at[pl.ds(base, _CHUNK)], rows_v)
            pltpu.sync_copy(rows_v, shared.at[idx_v], add=True)
            return carry

        lax.fori_loop(0, n_ch, body, 0)
        plsc.subcore_barrier()
        pltpu.sync_copy(shared.at[pl.ds(s * zc, zc)],
                        out_h.at[c, pl.ds(s * zc, zc)])

    return k(msg, dsts, zrows)


def kernel(x, params, edge_index):
    p = params
    n = x.shape[0]
    e = edge_index.shape[1]
    e_pad = _NW * _CHUNK * ((e + _NW * _CHUNK - 1) // (_NW * _CHUNK))
    src = edge_index[0].astype(jnp.int32)
    dst = edge_index[1].astype(jnp.int32)
    pad0 = jnp.zeros((e_pad - e,), jnp.int32)
    ia = jnp.concatenate([dst, pad0])
    ib = jnp.concatenate([src, pad0])
    dsts = jnp.concatenate([dst, jnp.full((e_pad - e,), n, jnp.int32)])
    zrows = jnp.zeros((_N_ACC, _D), jnp.float32)

    h = _node_mlp(x, p)
    h0 = h
    for lp in p['gnn']:
        ai, aj = _proj(h, lp['Wm'][:, :_D], lp['Wm'][:, _D:])
        ga, gb = _sc_gather2(ai, aj, ia, ib)
        msg = _msg(ga, gb, lp)
        agg2 = _sc_scatter_add(msg, dsts, zrows)
        h = _update(h, agg2, lp)

    pfin = _mm2(h, h0, p['fin_W1'])
    ga, gb = _sc_gather2(pfin, pfin, ia, ib)
    st = _stats(ga, gb)
    out8 = _final(ga, gb, st, p, e)
    return out8[:e, :3]
